# baseline (speedup 1.0000x reference)
import jax
import jax.numpy as jnp
from jax.experimental import pallas as pl
from jax.experimental.pallas import tpu as pltpu

_B, _C, _HW = 16, 64, 576
_K = 1024
_NTOT = _B * _HW
_PB = 4
_N2 = _PB * _HW
_G = _B // _PB


def _vq_body(x_ref, cb_ref, enc_ref, q_ref, loss_ref, perp_ref,
             counts_ref, acc_ref):
    s = pl.program_id(0)
    cb = cb_ref[...]
    xT = jnp.concatenate([x_ref[i].T for i in range(_PB)], axis=0)
    p2 = jnp.dot(xT, -2.0 * cb, preferred_element_type=jnp.float32)
    xsq = jnp.sum(xT * xT, axis=1, keepdims=True)
    csq = jnp.sum(cb * cb, axis=0, keepdims=True)
    d = (xsq + p2) + csq

    m = jnp.min(d, axis=1, keepdims=True)
    maskb = d == m
    mask = maskb.astype(jnp.float32)
    enc_ref[...] = mask

    qt = jax.lax.dot_general(cb, mask, (((1,), (1,)), ((), ())),
                             preferred_element_type=jnp.float32)
    for i in range(_PB):
        q_ref[i] = qt[:, i * _HW:(i + 1) * _HW]

    xcat = jnp.concatenate([x_ref[i] for i in range(_PB)], axis=1)
    diff = qt - xcat
    part = jnp.sum(diff * diff)
    cnt = jnp.sum(mask, axis=0, keepdims=True)

    @pl.when(s == 0)
    def _():
        acc_ref[0] = part
        counts_ref[...] = cnt

    @pl.when(s != 0)
    def _():
        acc_ref[0] = acc_ref[0] + part
        counts_ref[...] = counts_ref[...] + cnt

    tie = jnp.sum(cnt) > float(_N2) + 0.5

    @pl.when(tie)
    def _():
        lio = jax.lax.broadcasted_iota(jnp.int32, (_N2, _K), 1)
        idx = jnp.min(jnp.where(maskb, lio, _K), axis=1, keepdims=True)
        encf = (lio == idx).astype(jnp.float32)
        enc_ref[...] = encf
        q2 = jax.lax.dot_general(cb, encf, (((1,), (1,)), ((), ())),
                                 preferred_element_type=jnp.float32)
        for i in range(_PB):
            q_ref[i] = q2[:, i * _HW:(i + 1) * _HW]
        d2 = q2 - xcat
        part2 = jnp.sum(d2 * d2)
        cnt2 = jnp.sum(encf, axis=0, keepdims=True)
        acc_ref[0] = acc_ref[0] + (part2 - part)
        counts_ref[...] = counts_ref[...] + (cnt2 - cnt)

    @pl.when(s == _G - 1)
    def _():
        loss_ref[0] = 1.25 * acc_ref[0] / float(_NTOT * _C)
        p = counts_ref[...] / float(_NTOT)
        ent = jnp.sum(p * jnp.log(p + 1e-10))
        perp_ref[0] = jnp.exp(-ent)


@jax.jit
def kernel(x, codebook):
    xr = x.reshape(_B, _C, _HW)
    enc, q, loss, perp = pl.pallas_call(
        _vq_body,
        grid=(_G,),
        in_specs=[
            pl.BlockSpec((_PB, _C, _HW), lambda s: (s, 0, 0)),
            pl.BlockSpec((_C, _K), lambda s: (0, 0)),
        ],
        out_specs=[
            pl.BlockSpec((_N2, _K), lambda s: (s, 0)),
            pl.BlockSpec((_PB, _C, _HW), lambda s: (s, 0, 0)),
            pl.BlockSpec(memory_space=pltpu.SMEM),
            pl.BlockSpec(memory_space=pltpu.SMEM),
        ],
        out_shape=[
            jax.ShapeDtypeStruct((_NTOT, _K), jnp.float32),
            jax.ShapeDtypeStruct((_B, _C, _HW), jnp.float32),
            jax.ShapeDtypeStruct((1,), jnp.float32),
            jax.ShapeDtypeStruct((1,), jnp.float32),
        ],
        scratch_shapes=[
            pltpu.VMEM((1, _K), jnp.float32),
            pltpu.SMEM((1,), jnp.float32),
        ],
        compiler_params=pltpu.CompilerParams(
            dimension_semantics=("arbitrary",)),
    )(xr, codebook)
    return (loss[0], q.reshape(16, 64, 24, 24), perp[0], enc)
